# baseline (speedup 1.0000x reference)
import functools

import jax
import jax.numpy as jnp
from jax import lax
from jax.experimental import pallas as pl
from jax.experimental.pallas import tpu as pltpu
from jax.experimental.pallas import tpu_sc as plsc

_T = 11
_SENT = 11
_NC = 2
_NS = 16
_LANES = 16
_ZE = 16384
_ZSIZES = (16384, 8192, 4096, 2048, 1024, 512, 256, 128, 64, 32, 16, 8)


def _mod11(s):
  s = jnp.where(s >= 44, s - 44, s)
  s = jnp.where(s >= 22, s - 22, s)
  return jnp.where(s >= 11, s - 11, s)


def _sc_body(coords_hbm, resn_hbm, atmn_hbm, na_hbm, out_hbm, hist_hbm,
             resn_v, atmn_v, coords_v, tp_v, packed_v, offs_v,
             zeros_v, na_v, counts_v, heads_v, rags_v, sem_z, sem_s):
  B, N = resn_hbm.shape
  N3 = 3 * N
  REG3 = _T * N3
  RPW = B // (_NC * _NS)

  wid = lax.axis_index("s") * _NC + lax.axis_index("c")
  iota = lax.iota(jnp.int32, _LANES)

  pltpu.sync_copy(na_hbm, na_v)

  @pl.loop(0, _ZE // _LANES)
  def _zinit(i):
    zeros_v[pl.ds(i * _LANES, _LANES)] = jnp.zeros((_LANES,), jnp.float32)

  for r in range(RPW):
    b = wid * RPW + r
    row_base3 = b * REG3

    counts_v[...] = jnp.zeros((_LANES,), jnp.int32)
    pltpu.sync_copy(
        (resn_hbm.at[b], atmn_hbm.at[b], coords_hbm.at[b]),
        (resn_v, atmn_v, coords_v),
    )

    na_splat = plsc.load_gather(na_v, [jnp.full((_LANES,), b, jnp.int32)])
    n_a = na_splat[0]
    nblk = (n_a + 127) // 128

    @pl.loop(0, nblk)
    def _pass1(j):
      for k in range(8):
        i = j * 128 + k * _LANES
        t = _mod11(resn_v[pl.ds(i, _LANES)] + atmn_v[pl.ds(i, _LANES)])
        valid = (i + iota) < na_splat
        t = jnp.where(valid, t, _SENT)
        base = plsc.load_gather(counts_v, [t])
        rank, last = plsc.scan_count(t)
        plsc.addupdate_scatter(counts_v, [t], rank, mask=last)
        tp_v[pl.ds(i, _LANES)] = t * N + (base + rank - 1)

    cvec = counts_v[...]
    pc3 = ((cvec * 3 + 7) >> 3) << 3
    seg3 = plsc.cumsum(pc3) - pc3
    offs_v[...] = seg3

    @pl.loop(0, nblk)
    def _pass2(j):
      for k in range(8):
        i = j * 128 + k * _LANES
        v = tp_v[pl.ds(i, _LANES)]
        t = lax.shift_right_logical(v, N.bit_length() - 1)
        pos = v & (N - 1)
        e = plsc.load_gather(offs_v, [t]) + pos * 3
        a3 = (i + iota) * 3
        plsc.store_scatter(packed_v, [e], plsc.load_gather(coords_v, [a3]))
        plsc.store_scatter(packed_v, [e + 1],
                           plsc.load_gather(coords_v, [a3 + 1]))
        plsc.store_scatter(packed_v, [e + 2],
                           plsc.load_gather(coords_v, [a3 + 2]))

    def _out_dmas(issue):
      for t in range(_T):
        cnt3 = cvec[t] * 3
        dst0 = pl.multiple_of(row_base3 + t * N3, 8)
        src0 = pl.multiple_of(seg3[t], 8)
        l8 = (cnt3 >> 3) << 3
        rag = cnt3 & 7

        off_s, off_d = src0, dst0
        for size in _ZSIZES:
          cond = (l8 & size) != 0

          @pl.when(cond)
          def _():
            if issue:
              pltpu.async_copy(packed_v.at[pl.ds(off_s, size)],
                               out_hbm.at[pl.ds(off_d, size)], sem_s)
            else:
              pltpu.make_async_copy(packed_v.at[pl.ds(off_s, size)],
                                    out_hbm.at[pl.ds(off_d, size)],
                                    sem_s).wait()

          off_s = pl.multiple_of(off_s + jnp.where(cond, size, 0), 8)
          off_d = pl.multiple_of(off_d + jnp.where(cond, size, 0), 8)

        if issue:
          rags_v[t, :] = jnp.where(iota < rag, off_d + iota, -1)
          pltpu.async_copy(
              packed_v.at[pl.ds(off_s, _LANES)],
              out_hbm.at[plsc.Indices(rags_v.at[t], ignored_value=-1)], sem_s)
        else:
          pltpu.make_async_copy(
              packed_v.at[pl.ds(off_s, _LANES)],
              out_hbm.at[plsc.Indices(rags_v.at[t], ignored_value=-1)],
              sem_s).wait()

        s0 = dst0 + cnt3
        end = row_base3 + (t + 1) * N3
        head = jnp.minimum((8 - (s0 % 8)) % 8, end - s0)
        if issue:
          heads_v[t, :] = s0 + jnp.where(iota < head, iota, 0)
          pltpu.async_copy(zeros_v.at[pl.ds(0, _LANES)],
                           out_hbm.at[heads_v.at[t]], sem_z)
        else:
          pltpu.make_async_copy(zeros_v.at[pl.ds(0, _LANES)],
                                out_hbm.at[heads_v.at[t]], sem_z).wait()
        off = pl.multiple_of(s0 + head, 8)
        rem = end - off
        for size in _ZSIZES:
          cond = (rem & size) != 0

          @pl.when(cond)
          def _():
            if issue:
              pltpu.async_copy(zeros_v.at[pl.ds(0, size)],
                               out_hbm.at[pl.ds(off, size)], sem_z)
            else:
              pltpu.make_async_copy(zeros_v.at[pl.ds(0, size)],
                                    out_hbm.at[pl.ds(off, size)], sem_z).wait()

          off = pl.multiple_of(off + jnp.where(cond, size, 0), 8)

    _out_dmas(issue=True)
    _out_dmas(issue=False)

    pltpu.sync_copy(counts_v, hist_hbm.at[b])


def kernel(input_coords_cpu, input_resnames, input_atomnames, num_atoms):
  B, N3 = input_coords_cpu.shape
  N = N3 // 3

  mesh = plsc.VectorSubcoreMesh(core_axis_name="c", subcore_axis_name="s",
                                num_cores=_NC, num_subcores=_NS)
  run = pl.kernel(
      _sc_body,
      out_type=(
          jax.ShapeDtypeStruct((B * _T * N3,), jnp.float32),
          jax.ShapeDtypeStruct((B, _LANES), jnp.int32),
      ),
      mesh=mesh,
      compiler_params=pltpu.CompilerParams(needs_layout_passes=False,
                                           use_tc_tiling_on_sc=False),
      scratch_types=[
          pltpu.VMEM((N,), jnp.int32),
          pltpu.VMEM((N,), jnp.int32),
          pltpu.VMEM((N3,), jnp.float32),
          pltpu.VMEM((N,), jnp.int32),
          pltpu.VMEM((N3 + 512,), jnp.float32),
          pltpu.VMEM((_LANES,), jnp.int32),
          pltpu.VMEM((_ZE,), jnp.float32),
          pltpu.VMEM((B,), jnp.int32),
          pltpu.VMEM((_LANES,), jnp.int32),
          pltpu.VMEM((_T, _LANES), jnp.int32),
          pltpu.VMEM((_T, _LANES), jnp.int32),
          pltpu.SemaphoreType.DMA,
          pltpu.SemaphoreType.DMA,
      ],
  )
  out, hist = run(input_coords_cpu, input_resnames, input_atomnames, num_atoms)
  return out.reshape(B, _T, N3), hist[:, :_T]


# speedup vs baseline: 1.0006x; 1.0006x over previous
import functools

import jax
import jax.numpy as jnp
from jax import lax
from jax.experimental import pallas as pl
from jax.experimental.pallas import tpu as pltpu
from jax.experimental.pallas import tpu_sc as plsc

_T = 11
_SENT = 11
_NC = 2
_NS = 16
_LANES = 16
_ZE = 16384
_ZSIZES = (16384, 8192, 4096, 2048, 1024, 512, 256, 128, 64, 32, 16, 8)


def _mod11(s):
  s = jnp.where(s >= 44, s - 44, s)
  s = jnp.where(s >= 22, s - 22, s)
  return jnp.where(s >= 11, s - 11, s)


def _sc_body(coords_hbm, resn_hbm, atmn_hbm, na_hbm, out_hbm, hist_hbm,
             resn_v, atmn_v, coords_v, tp_v, packed_v, offs_v,
             zeros_v, na_v, counts_v, heads_v, rags_v, sem_z, sem_s):
  B = na_hbm.shape[0]
  N = resn_hbm.shape[0] // B
  N3 = 3 * N
  REG3 = _T * N3
  RPW = B // (_NC * _NS)

  wid = lax.axis_index("s") * _NC + lax.axis_index("c")
  iota = lax.iota(jnp.int32, _LANES)

  pltpu.sync_copy(na_hbm, na_v)

  @pl.loop(0, _ZE // _LANES)
  def _zinit(i):
    zeros_v[pl.ds(i * _LANES, _LANES)] = jnp.zeros((_LANES,), jnp.float32)

  for r in range(RPW):
    b = wid * RPW + r
    row_base3 = b * REG3

    counts_v[...] = jnp.zeros((_LANES,), jnp.int32)
    pltpu.sync_copy(
        (resn_hbm.at[pl.ds(b * N, N)], atmn_hbm.at[pl.ds(b * N, N)],
         coords_hbm.at[pl.ds(b * N3, N3)]),
        (resn_v, atmn_v, coords_v),
    )

    na_splat = plsc.load_gather(na_v, [jnp.full((_LANES,), b, jnp.int32)])
    n_a = na_splat[0]
    nblk = (n_a + 127) // 128

    @pl.loop(0, nblk)
    def _pass1(j):
      for k in range(8):
        i = j * 128 + k * _LANES
        t = _mod11(resn_v[pl.ds(i, _LANES)] + atmn_v[pl.ds(i, _LANES)])
        valid = (i + iota) < na_splat
        t = jnp.where(valid, t, _SENT)
        base = plsc.load_gather(counts_v, [t])
        rank, last = plsc.scan_count(t)
        plsc.addupdate_scatter(counts_v, [t], rank, mask=last)
        tp_v[pl.ds(i, _LANES)] = t * N + (base + rank - 1)

    cvec = counts_v[...]
    pc3 = ((cvec * 3 + 7) >> 3) << 3
    seg3 = plsc.cumsum(pc3) - pc3
    offs_v[...] = seg3

    @pl.loop(0, nblk)
    def _pass2(j):
      for k in range(8):
        i = j * 128 + k * _LANES
        v = tp_v[pl.ds(i, _LANES)]
        t = lax.shift_right_logical(v, N.bit_length() - 1)
        pos = v & (N - 1)
        e = plsc.load_gather(offs_v, [t]) + pos * 3
        a3 = (i + iota) * 3
        plsc.store_scatter(packed_v, [e], plsc.load_gather(coords_v, [a3]))
        plsc.store_scatter(packed_v, [e + 1],
                           plsc.load_gather(coords_v, [a3 + 1]))
        plsc.store_scatter(packed_v, [e + 2],
                           plsc.load_gather(coords_v, [a3 + 2]))

    def _out_dmas(issue):
      for t in range(_T):
        cnt3 = cvec[t] * 3
        dst0 = pl.multiple_of(row_base3 + t * N3, 8)
        src0 = pl.multiple_of(seg3[t], 8)
        l8 = (cnt3 >> 3) << 3
        rag = cnt3 & 7

        off_s, off_d = src0, dst0
        for size in _ZSIZES:
          cond = (l8 & size) != 0

          @pl.when(cond)
          def _():
            if issue:
              pltpu.async_copy(packed_v.at[pl.ds(off_s, size)],
                               out_hbm.at[pl.ds(off_d, size)], sem_s)
            else:
              pltpu.make_async_copy(packed_v.at[pl.ds(off_s, size)],
                                    out_hbm.at[pl.ds(off_d, size)],
                                    sem_s).wait()

          off_s = pl.multiple_of(off_s + jnp.where(cond, size, 0), 8)
          off_d = pl.multiple_of(off_d + jnp.where(cond, size, 0), 8)

        if issue:
          rags_v[t, :] = jnp.where(iota < rag, off_d + iota, -1)
          pltpu.async_copy(
              packed_v.at[pl.ds(off_s, _LANES)],
              out_hbm.at[plsc.Indices(rags_v.at[t], ignored_value=-1)], sem_s)
        else:
          pltpu.make_async_copy(
              packed_v.at[pl.ds(off_s, _LANES)],
              out_hbm.at[plsc.Indices(rags_v.at[t], ignored_value=-1)],
              sem_s).wait()

        s0 = dst0 + cnt3
        end = row_base3 + (t + 1) * N3
        head = jnp.minimum((8 - (s0 % 8)) % 8, end - s0)
        if issue:
          heads_v[t, :] = s0 + jnp.where(iota < head, iota, 0)
          pltpu.async_copy(zeros_v.at[pl.ds(0, _LANES)],
                           out_hbm.at[heads_v.at[t]], sem_z)
        else:
          pltpu.make_async_copy(zeros_v.at[pl.ds(0, _LANES)],
                                out_hbm.at[heads_v.at[t]], sem_z).wait()
        off = pl.multiple_of(s0 + head, 8)
        rem = end - off
        for size in _ZSIZES:
          cond = (rem & size) != 0

          @pl.when(cond)
          def _():
            if issue:
              pltpu.async_copy(zeros_v.at[pl.ds(0, size)],
                               out_hbm.at[pl.ds(off, size)], sem_z)
            else:
              pltpu.make_async_copy(zeros_v.at[pl.ds(0, size)],
                                    out_hbm.at[pl.ds(off, size)], sem_z).wait()

          off = pl.multiple_of(off + jnp.where(cond, size, 0), 8)

    _out_dmas(issue=True)
    _out_dmas(issue=False)

    pltpu.sync_copy(counts_v, hist_hbm.at[pl.ds(b * _LANES, _LANES)])


def kernel(input_coords_cpu, input_resnames, input_atomnames, num_atoms):
  B, N3 = input_coords_cpu.shape
  N = N3 // 3

  mesh = plsc.VectorSubcoreMesh(core_axis_name="c", subcore_axis_name="s",
                                num_cores=_NC, num_subcores=_NS)
  run = pl.kernel(
      _sc_body,
      out_type=(
          jax.ShapeDtypeStruct((B * _T * N3,), jnp.float32),
          jax.ShapeDtypeStruct((B * _LANES,), jnp.int32),
      ),
      mesh=mesh,
      compiler_params=pltpu.CompilerParams(needs_layout_passes=False,
                                           use_tc_tiling_on_sc=False),
      scratch_types=[
          pltpu.VMEM((N,), jnp.int32),
          pltpu.VMEM((N,), jnp.int32),
          pltpu.VMEM((N3,), jnp.float32),
          pltpu.VMEM((N,), jnp.int32),
          pltpu.VMEM((N3 + 512,), jnp.float32),
          pltpu.VMEM((_LANES,), jnp.int32),
          pltpu.VMEM((_ZE,), jnp.float32),
          pltpu.VMEM((B,), jnp.int32),
          pltpu.VMEM((_LANES,), jnp.int32),
          pltpu.VMEM((_T, _LANES), jnp.int32),
          pltpu.VMEM((_T, _LANES), jnp.int32),
          pltpu.SemaphoreType.DMA,
          pltpu.SemaphoreType.DMA,
      ],
  )
  out, hist = run(input_coords_cpu.reshape(B * N3),
                  input_resnames.reshape(B * N),
                  input_atomnames.reshape(B * N), num_atoms)
  return out.reshape(B, _T, N3), hist.reshape(B, _LANES)[:, :_T]


# speedup vs baseline: 1.0071x; 1.0064x over previous
import functools

import jax
import jax.numpy as jnp
from jax import lax
from jax.experimental import pallas as pl
from jax.experimental.pallas import tpu as pltpu
from jax.experimental.pallas import tpu_sc as plsc

_T = 11
_SENT = 11
_NC = 2
_NS = 16
_LANES = 16
_ZE = 16384
_ZSIZES = (16384, 8192, 4096, 2048, 1024, 512, 256, 128, 64, 32, 16, 8)


def _mod11(s):
  s = jnp.where(s >= 44, s - 44, s)
  s = jnp.where(s >= 22, s - 22, s)
  return jnp.where(s >= 11, s - 11, s)


def _sc_body(coords_hbm, resn_hbm, atmn_hbm, na_hbm, out_hbm, hist_hbm,
             resn_v, atmn_v, coords_v, tp_v, packed0_v, packed1_v, offs_v,
             zeros_v, na_v, counts_v, heads0_v, heads1_v, rags0_v, rags1_v,
             sem_z, sem_s):
  B = na_hbm.shape[0]
  N = resn_hbm.shape[0] // B
  N3 = 3 * N
  REG3 = _T * N3
  RPW = B // (_NC * _NS)

  wid = lax.axis_index("s") * _NC + lax.axis_index("c")
  iota = lax.iota(jnp.int32, _LANES)

  pltpu.sync_copy(na_hbm, na_v)

  @pl.loop(0, _ZE // _LANES)
  def _zinit(i):
    zeros_v[pl.ds(i * _LANES, _LANES)] = jnp.zeros((_LANES,), jnp.float32)

  drain_prev = None
  for r in range(RPW):
    packed_v = (packed0_v, packed1_v)[r % 2]
    heads_v = (heads0_v, heads1_v)[r % 2]
    rags_v = (rags0_v, rags1_v)[r % 2]
    b = wid * RPW + r
    row_base3 = b * REG3

    counts_v[...] = jnp.zeros((_LANES,), jnp.int32)
    pltpu.sync_copy(
        (resn_hbm.at[pl.ds(b * N, N)], atmn_hbm.at[pl.ds(b * N, N)],
         coords_hbm.at[pl.ds(b * N3, N3)]),
        (resn_v, atmn_v, coords_v),
    )

    na_splat = plsc.load_gather(na_v, [jnp.full((_LANES,), b, jnp.int32)])
    n_a = na_splat[0]
    nblk = (n_a + 127) // 128

    @pl.loop(0, nblk)
    def _pass1(j):
      for k in range(8):
        i = j * 128 + k * _LANES
        t = _mod11(resn_v[pl.ds(i, _LANES)] + atmn_v[pl.ds(i, _LANES)])
        valid = (i + iota) < na_splat
        t = jnp.where(valid, t, _SENT)
        base = plsc.load_gather(counts_v, [t])
        rank, last = plsc.scan_count(t)
        plsc.addupdate_scatter(counts_v, [t], rank, mask=last)
        tp_v[pl.ds(i, _LANES)] = t * N + (base + rank - 1)

    cvec = counts_v[...]
    pc3 = ((cvec * 3 + 7) >> 3) << 3
    seg3 = plsc.cumsum(pc3) - pc3
    offs_v[...] = seg3

    @pl.loop(0, nblk)
    def _pass2(j):
      for k in range(8):
        i = j * 128 + k * _LANES
        v = tp_v[pl.ds(i, _LANES)]
        t = lax.shift_right_logical(v, N.bit_length() - 1)
        pos = v & (N - 1)
        e = plsc.load_gather(offs_v, [t]) + pos * 3
        a3 = (i + iota) * 3
        plsc.store_scatter(packed_v, [e], plsc.load_gather(coords_v, [a3]))
        plsc.store_scatter(packed_v, [e + 1],
                           plsc.load_gather(coords_v, [a3 + 1]))
        plsc.store_scatter(packed_v, [e + 2],
                           plsc.load_gather(coords_v, [a3 + 2]))

    def _out_dmas(issue, cvec=cvec, seg3=seg3, row_base3=row_base3,
                  packed_v=packed_v, heads_v=heads_v, rags_v=rags_v):
      for t in range(_T):
        cnt3 = cvec[t] * 3
        dst0 = pl.multiple_of(row_base3 + t * N3, 8)
        src0 = pl.multiple_of(seg3[t], 8)
        l8 = (cnt3 >> 3) << 3
        rag = cnt3 & 7

        off_s, off_d = src0, dst0
        for size in _ZSIZES:
          cond = (l8 & size) != 0

          @pl.when(cond)
          def _():
            if issue:
              pltpu.async_copy(packed_v.at[pl.ds(off_s, size)],
                               out_hbm.at[pl.ds(off_d, size)], sem_s)
            else:
              pltpu.make_async_copy(packed_v.at[pl.ds(off_s, size)],
                                    out_hbm.at[pl.ds(off_d, size)],
                                    sem_s).wait()

          off_s = pl.multiple_of(off_s + jnp.where(cond, size, 0), 8)
          off_d = pl.multiple_of(off_d + jnp.where(cond, size, 0), 8)

        if issue:
          rags_v[t, :] = jnp.where(iota < rag, off_d + iota, -1)
          pltpu.async_copy(
              packed_v.at[pl.ds(off_s, _LANES)],
              out_hbm.at[plsc.Indices(rags_v.at[t], ignored_value=-1)], sem_s)
        else:
          pltpu.make_async_copy(
              packed_v.at[pl.ds(off_s, _LANES)],
              out_hbm.at[plsc.Indices(rags_v.at[t], ignored_value=-1)],
              sem_s).wait()

        s0 = dst0 + cnt3
        end = row_base3 + (t + 1) * N3
        head = jnp.minimum((8 - (s0 % 8)) % 8, end - s0)
        if issue:
          heads_v[t, :] = s0 + jnp.where(iota < head, iota, 0)
          pltpu.async_copy(zeros_v.at[pl.ds(0, _LANES)],
                           out_hbm.at[heads_v.at[t]], sem_z)
        else:
          pltpu.make_async_copy(zeros_v.at[pl.ds(0, _LANES)],
                                out_hbm.at[heads_v.at[t]], sem_z).wait()
        off = pl.multiple_of(s0 + head, 8)
        rem = end - off
        for size in _ZSIZES:
          cond = (rem & size) != 0

          @pl.when(cond)
          def _():
            if issue:
              pltpu.async_copy(zeros_v.at[pl.ds(0, size)],
                               out_hbm.at[pl.ds(off, size)], sem_z)
            else:
              pltpu.make_async_copy(zeros_v.at[pl.ds(0, size)],
                                    out_hbm.at[pl.ds(off, size)], sem_z).wait()

          off = pl.multiple_of(off + jnp.where(cond, size, 0), 8)

    _out_dmas(issue=True)
    if drain_prev is not None:
      drain_prev()
    drain_prev = functools.partial(_out_dmas, issue=False)

    pltpu.sync_copy(counts_v, hist_hbm.at[pl.ds(b * _LANES, _LANES)])

  drain_prev()


def kernel(input_coords_cpu, input_resnames, input_atomnames, num_atoms):
  B, N3 = input_coords_cpu.shape
  N = N3 // 3

  mesh = plsc.VectorSubcoreMesh(core_axis_name="c", subcore_axis_name="s",
                                num_cores=_NC, num_subcores=_NS)
  run = pl.kernel(
      _sc_body,
      out_type=(
          jax.ShapeDtypeStruct((B * _T * N3,), jnp.float32),
          jax.ShapeDtypeStruct((B * _LANES,), jnp.int32),
      ),
      mesh=mesh,
      compiler_params=pltpu.CompilerParams(needs_layout_passes=False,
                                           use_tc_tiling_on_sc=False),
      scratch_types=[
          pltpu.VMEM((N,), jnp.int32),
          pltpu.VMEM((N,), jnp.int32),
          pltpu.VMEM((N3,), jnp.float32),
          pltpu.VMEM((N,), jnp.int32),
          pltpu.VMEM((N3 + 512,), jnp.float32),
          pltpu.VMEM((N3 + 512,), jnp.float32),
          pltpu.VMEM((_LANES,), jnp.int32),
          pltpu.VMEM((_ZE,), jnp.float32),
          pltpu.VMEM((B,), jnp.int32),
          pltpu.VMEM((_LANES,), jnp.int32),
          pltpu.VMEM((_T, _LANES), jnp.int32),
          pltpu.VMEM((_T, _LANES), jnp.int32),
          pltpu.VMEM((_T, _LANES), jnp.int32),
          pltpu.VMEM((_T, _LANES), jnp.int32),
          pltpu.SemaphoreType.DMA,
          pltpu.SemaphoreType.DMA,
      ],
  )
  out, hist = run(input_coords_cpu.reshape(B * N3),
                  input_resnames.reshape(B * N),
                  input_atomnames.reshape(B * N), num_atoms)
  return out.reshape(B, _T, N3), hist.reshape(B, _LANES)[:, :_T]


# speedup vs baseline: 1.0658x; 1.0583x over previous
import functools

import jax
import jax.numpy as jnp
from jax import lax
from jax.experimental import pallas as pl
from jax.experimental.pallas import tpu as pltpu
from jax.experimental.pallas import tpu_sc as plsc

_T = 11
_SENT = 11
_NC = 2
_NS = 16
_LANES = 16
_ZE = 16384
_ZSIZES = (16384, 8192, 4096, 2048, 1024, 512, 256, 128, 64, 32, 16, 8)


def _mod11(s):
  s = jnp.where(s >= 44, s - 44, s)
  s = jnp.where(s >= 22, s - 22, s)
  return jnp.where(s >= 11, s - 11, s)


def _sc_body(coords_hbm, resn_hbm, atmn_hbm, na_hbm, out_hbm, hist_hbm,
             resn_v, atmn_v, coords_v, tp_v, packed0_v, packed1_v, offs_v,
             zeros_v, na_v, counts_v, heads0_v, heads1_v, rags0_v, rags1_v,
             sem_z, sem_s):
  B = na_hbm.shape[0]
  N = resn_hbm.shape[0] // B
  N3 = 3 * N
  REG3 = _T * N3
  RPW = B // (_NC * _NS)

  wid = lax.axis_index("s") * _NC + lax.axis_index("c")
  iota = lax.iota(jnp.int32, _LANES)

  pltpu.sync_copy(na_hbm, na_v)

  @pl.loop(0, _ZE // _LANES)
  def _zinit(i):
    zeros_v[pl.ds(i * _LANES, _LANES)] = jnp.zeros((_LANES,), jnp.float32)

  drain_prev = None
  for r in range(RPW):
    packed_v = (packed0_v, packed1_v)[r % 2]
    heads_v = (heads0_v, heads1_v)[r % 2]
    rags_v = (rags0_v, rags1_v)[r % 2]
    b = wid * RPW + r
    row_base3 = b * REG3

    counts_v[...] = jnp.zeros((_LANES,), jnp.int32)
    pltpu.sync_copy(
        (resn_hbm.at[pl.ds(b * N, N)], atmn_hbm.at[pl.ds(b * N, N)],
         coords_hbm.at[pl.ds(b * N3, N3)]),
        (resn_v, atmn_v, coords_v),
    )

    na_splat = plsc.load_gather(na_v, [jnp.full((_LANES,), b, jnp.int32)])
    n_a = na_splat[0]
    nblk = (n_a + 127) // 128

    @pl.loop(0, nblk)
    def _pass1(j):
      for k in range(8):
        i = j * 128 + k * _LANES
        t = _mod11(resn_v[pl.ds(i, _LANES)] + atmn_v[pl.ds(i, _LANES)])
        valid = (i + iota) < na_splat
        t = jnp.where(valid, t, _SENT)
        base = plsc.load_gather(counts_v, [t])
        rank, last = plsc.scan_count(t)
        plsc.addupdate_scatter(counts_v, [t], rank, mask=last)
        tp_v[pl.ds(i, _LANES)] = t * N + (base + rank - 1)

    cvec = counts_v[...]
    pc3 = ((cvec * 3 + 7) >> 3) << 3
    seg3 = plsc.cumsum(pc3) - pc3
    offs_v[...] = seg3

    @pl.loop(0, nblk)
    def _pass2(j):
      for k in range(8):
        i = j * 128 + k * _LANES
        v = tp_v[pl.ds(i, _LANES)]
        t = lax.shift_right_logical(v, N.bit_length() - 1)
        pos = v & (N - 1)
        e = plsc.load_gather(offs_v, [t]) + pos * 3
        a3 = (i + iota) * 3
        plsc.store_scatter(packed_v, [e], plsc.load_gather(coords_v, [a3]))
        plsc.store_scatter(packed_v, [e + 1],
                           plsc.load_gather(coords_v, [a3 + 1]))
        plsc.store_scatter(packed_v, [e + 2],
                           plsc.load_gather(coords_v, [a3 + 2]))

    def _out_dmas(issue, cvec=cvec, seg3=seg3, row_base3=row_base3,
                  packed_v=packed_v, heads_v=heads_v, rags_v=rags_v):
      for t in range(_T):
        cnt3 = cvec[t] * 3
        dst0 = pl.multiple_of(row_base3 + t * N3, 8)
        src0 = pl.multiple_of(seg3[t], 8)
        l8 = (cnt3 >> 3) << 3
        rag = cnt3 & 7

        off_s, off_d = src0, dst0
        for size in _ZSIZES:
          cond = (l8 & size) != 0

          @pl.when(cond)
          def _():
            if issue:
              pltpu.async_copy(packed_v.at[pl.ds(off_s, size)],
                               out_hbm.at[pl.ds(off_d, size)], sem_s)
            else:
              pltpu.make_async_copy(packed_v.at[pl.ds(off_s, size)],
                                    out_hbm.at[pl.ds(off_d, size)],
                                    sem_s).wait()

          off_s = pl.multiple_of(off_s + jnp.where(cond, size, 0), 8)
          off_d = pl.multiple_of(off_d + jnp.where(cond, size, 0), 8)

        if issue:
          rags_v[t, :] = jnp.where(iota < rag, off_d + iota, -1)
          pltpu.async_copy(
              packed_v.at[pl.ds(off_s, _LANES)],
              out_hbm.at[plsc.Indices(rags_v.at[t], ignored_value=-1)], sem_s)
        else:
          pltpu.make_async_copy(
              packed_v.at[pl.ds(off_s, _LANES)],
              out_hbm.at[plsc.Indices(rags_v.at[t], ignored_value=-1)],
              sem_s).wait()

        s0 = dst0 + cnt3
        end = row_base3 + (t + 1) * N3
        head = jnp.minimum((8 - (s0 % 8)) % 8, end - s0)
        if issue:
          heads_v[t, :] = jnp.where(iota < head, s0 + iota, -1)
          pltpu.async_copy(
              zeros_v.at[pl.ds(0, _LANES)],
              out_hbm.at[plsc.Indices(heads_v.at[t], ignored_value=-1)], sem_z)
        else:
          pltpu.make_async_copy(
              zeros_v.at[pl.ds(0, _LANES)],
              out_hbm.at[plsc.Indices(heads_v.at[t], ignored_value=-1)],
              sem_z).wait()
        off = pl.multiple_of(s0 + head, 8)
        rem = end - off
        for size in _ZSIZES:
          cond = (rem & size) != 0

          @pl.when(cond)
          def _():
            if issue:
              pltpu.async_copy(zeros_v.at[pl.ds(0, size)],
                               out_hbm.at[pl.ds(off, size)], sem_z)
            else:
              pltpu.make_async_copy(zeros_v.at[pl.ds(0, size)],
                                    out_hbm.at[pl.ds(off, size)], sem_z).wait()

          off = pl.multiple_of(off + jnp.where(cond, size, 0), 8)

    _out_dmas(issue=True)
    if drain_prev is not None:
      drain_prev()
    drain_prev = functools.partial(_out_dmas, issue=False)

    pltpu.sync_copy(counts_v, hist_hbm.at[pl.ds(b * _LANES, _LANES)])

  drain_prev()


def kernel(input_coords_cpu, input_resnames, input_atomnames, num_atoms):
  B, N3 = input_coords_cpu.shape
  N = N3 // 3

  mesh = plsc.VectorSubcoreMesh(core_axis_name="c", subcore_axis_name="s",
                                num_cores=_NC, num_subcores=_NS)
  run = pl.kernel(
      _sc_body,
      out_type=(
          jax.ShapeDtypeStruct((B * _T * N3,), jnp.float32),
          jax.ShapeDtypeStruct((B * _LANES,), jnp.int32),
      ),
      mesh=mesh,
      compiler_params=pltpu.CompilerParams(needs_layout_passes=False,
                                           use_tc_tiling_on_sc=False),
      scratch_types=[
          pltpu.VMEM((N,), jnp.int32),
          pltpu.VMEM((N,), jnp.int32),
          pltpu.VMEM((N3,), jnp.float32),
          pltpu.VMEM((N,), jnp.int32),
          pltpu.VMEM((N3 + 512,), jnp.float32),
          pltpu.VMEM((N3 + 512,), jnp.float32),
          pltpu.VMEM((_LANES,), jnp.int32),
          pltpu.VMEM((_ZE,), jnp.float32),
          pltpu.VMEM((B,), jnp.int32),
          pltpu.VMEM((_LANES,), jnp.int32),
          pltpu.VMEM((_T, _LANES), jnp.int32),
          pltpu.VMEM((_T, _LANES), jnp.int32),
          pltpu.VMEM((_T, _LANES), jnp.int32),
          pltpu.VMEM((_T, _LANES), jnp.int32),
          pltpu.SemaphoreType.DMA,
          pltpu.SemaphoreType.DMA,
      ],
  )
  out, hist = run(input_coords_cpu.reshape(B * N3),
                  input_resnames.reshape(B * N),
                  input_atomnames.reshape(B * N), num_atoms)
  return out.reshape(B, _T, N3), hist.reshape(B, _LANES)[:, :_T]
